# Initial kernel scaffold; baseline (speedup 1.0000x reference)
import jax
import jax.numpy as jnp
from jax.experimental import pallas as pl


def kernel(user_ids, item_ids, user_table, item_table, W1, b1, W2, b2, W3, b3):
    raise NotImplementedError("write your pallas kernel here")


# speedup vs baseline: 5.0681x; 5.0681x over previous
import functools

import jax
import jax.numpy as jnp
from jax import lax
from jax.experimental import pallas as pl
from jax.experimental.pallas import tpu as pltpu
from jax.experimental.pallas import tpu_sc as plsc

NC = 2
NS = 16
NW = NC * NS
CHUNK = 128


def _sc_gather_body(n_chunks, emb, user_table, item_table, uids, iids,
                    out_u, out_i, uidx_v, iidx_v, urows, irows, sem):
    wid = lax.axis_index("s") * NC + lax.axis_index("c")
    rbase = wid * n_chunks
    pltpu.sync_copy(uids.at[pl.ds(rbase, n_chunks)], uidx_v)
    pltpu.sync_copy(iids.at[pl.ds(rbase, n_chunks)], iidx_v)
    copies = []
    for j in range(n_chunks):
        copies.append(pltpu.async_copy(
            user_table.at[uidx_v.at[j]], urows.at[pl.ds(j * CHUNK, CHUNK)], sem))
        copies.append(pltpu.async_copy(
            item_table.at[iidx_v.at[j]], irows.at[pl.ds(j * CHUNK, CHUNK)], sem))
    for cp in copies:
        cp.wait()
    base = wid * n_chunks * CHUNK
    pltpu.sync_copy(urows, out_u.at[pl.ds(base, n_chunks * CHUNK)])
    pltpu.sync_copy(irows, out_i.at[pl.ds(base, n_chunks * CHUNK)])


@functools.partial(jax.jit, static_argnums=(0, 1))
def _sc_gather(n_chunks, emb, user_table, item_table, uids2, iids2):
    batch = n_chunks * CHUNK * NW
    per_w = n_chunks * CHUNK
    mesh = plsc.VectorSubcoreMesh(core_axis_name="c", subcore_axis_name="s")
    fn = pl.kernel(
        functools.partial(_sc_gather_body, n_chunks, emb),
        out_type=(
            jax.ShapeDtypeStruct((batch, emb), jnp.float32),
            jax.ShapeDtypeStruct((batch, emb), jnp.float32),
        ),
        mesh=mesh,
        scratch_types=[
            pltpu.VMEM((n_chunks, CHUNK), jnp.int32),
            pltpu.VMEM((n_chunks, CHUNK), jnp.int32),
            pltpu.VMEM((per_w, emb), jnp.float32),
            pltpu.VMEM((per_w, emb), jnp.float32),
            pltpu.SemaphoreType.DMA,
        ],
    )
    return fn(user_table, item_table, uids2, iids2)


def _mlp_body(u_ref, v_ref, w1u_ref, w1i_ref, b1_ref, w2_ref, b2_ref,
              w3_ref, b3_ref, out_ref):
    dot = functools.partial(
        jnp.dot, preferred_element_type=jnp.float32,
        precision=lax.Precision.HIGHEST)
    h1 = (dot(u_ref[...], w1u_ref[...]) + dot(v_ref[...], w1i_ref[...])
          + b1_ref[...])
    h1 = jnp.where(h1 >= 0, h1, 0.01 * h1)
    h2 = dot(h1, w2_ref[...]) + b2_ref[...]
    h2 = jnp.where(h2 >= 0, h2, 0.01 * h2)
    out_ref[...] = dot(h2, w3_ref[...]) + b3_ref[...]


@functools.partial(jax.jit, static_argnums=(0,))
def _mlp(block_rows, u_emb, v_emb, w1u, w1i, b1, w2, b2, w3, b3):
    batch, emb = u_emb.shape
    d1 = w1u.shape[1]
    d2 = w2.shape[1]
    grid = (batch // block_rows,)
    full = lambda shape: pl.BlockSpec(shape, lambda i: (0, 0))
    return pl.pallas_call(
        _mlp_body,
        grid=grid,
        in_specs=[
            pl.BlockSpec((block_rows, emb), lambda i: (i, 0)),
            pl.BlockSpec((block_rows, emb), lambda i: (i, 0)),
            full((emb, d1)),
            full((emb, d1)),
            full((1, d1)),
            full((d1, d2)),
            full((1, d2)),
            full((d2, 1)),
            full((1, 1)),
        ],
        out_specs=pl.BlockSpec((block_rows, 1), lambda i: (i, 0)),
        out_shape=jax.ShapeDtypeStruct((batch, 1), jnp.float32),
    )(u_emb, v_emb, w1u, w1i, b1, w2, b2, w3, b3)


def kernel(user_ids, item_ids, user_table, item_table, W1, b1, W2, b2, W3, b3):
    batch = user_ids.shape[0]
    emb = user_table.shape[1]
    u_emb = jnp.take(user_table, user_ids, axis=0)
    v_emb = jnp.take(item_table, item_ids, axis=0)
    w1t = W1.T
    w1u, w1i = w1t[:emb], w1t[emb:]
    y = _mlp(2048, u_emb, v_emb, w1u, w1i, b1.reshape(1, -1), W2.T,
             b2.reshape(1, -1), W3.T, b3.reshape(1, 1))
    return y.reshape(batch)
